# baseline (speedup 1.0000x reference)
import functools

import jax
import jax.numpy as jnp
from jax import lax
from jax.experimental import pallas as pl
from jax.experimental.pallas import tpu as pltpu
from jax.experimental.pallas import tpu_sc as plsc

NUM_CORES = 2
NUM_SUBCORES = 16
NW = NUM_CORES * NUM_SUBCORES

ROWS_PER_CHUNK = 8


def _make_gather(N, H, V, D):
    B = N * H
    assert N % (NW * 2 * ROWS_PER_CHUNK) == 0
    rows_per_w = N // NW
    n_chunks = rows_per_w // ROWS_PER_CHUNK
    chunk = ROWS_PER_CHUNK * H

    mesh = plsc.VectorSubcoreMesh(
        core_axis_name="c", subcore_axis_name="s", num_cores=NUM_CORES
    )

    @functools.partial(
        pl.kernel,
        mesh=mesh,
        out_type=jax.ShapeDtypeStruct((N, H, D), jnp.float32),
        scratch_types=[
            pltpu.VMEM((2, chunk), jnp.int32),
            pltpu.VMEM((2, ROWS_PER_CHUNK, H, D), jnp.float32),
            pltpu.SemaphoreType.DMA((2,)),
            pltpu.SemaphoreType.DMA((2, ROWS_PER_CHUNK)),
            pltpu.SemaphoreType.DMA((2,)),
        ],
        compiler_params=pltpu.CompilerParams(use_tc_tiling_on_sc=False),
    )
    def gather_kernel(idx_hbm, tbl_hbm, out_hbm, idx_v, rows_v, isem, gsem, osem):
        wid = lax.axis_index("s") * NUM_CORES + lax.axis_index("c")
        w_base = wid * rows_per_w * H
        r_base = wid * rows_per_w

        def idx_copy(j, b):
            return pltpu.make_async_copy(
                idx_hbm.at[pl.ds(w_base + j * chunk, chunk)], idx_v.at[b], isem.at[b]
            )

        def gather_copy(b, k):
            return pltpu.make_async_copy(
                tbl_hbm.at[idx_v.at[b, pl.ds(k * H, H)]],
                rows_v.at[b, k],
                gsem.at[b, k],
            )

        def out_copy(j, b):
            return pltpu.make_async_copy(
                rows_v.at[b],
                out_hbm.at[pl.ds(r_base + j * ROWS_PER_CHUNK, ROWS_PER_CHUNK)],
                osem.at[b],
            )

        idx_copy(0, 0).start()
        idx_copy(1, 1).start()

        def body(t, carry):
            for b in (0, 1):
                j = 2 * t + b
                idx_copy(j, b).wait()

                @pl.when(j >= 2)
                def _():
                    out_copy(j - 2, b).wait()

                for k in range(ROWS_PER_CHUNK):
                    gather_copy(b, k).start()
                for k in range(ROWS_PER_CHUNK):
                    gather_copy(b, k).wait()

                @pl.when(j + 2 < n_chunks)
                def _():
                    idx_copy(j + 2, b).start()

                out_copy(j, b).start()
            return carry

        lax.fori_loop(0, n_chunks // 2, body, 0)
        out_copy(n_chunks - 2, 0).wait()
        out_copy(n_chunks - 1, 1).wait()

    return gather_kernel


def kernel(x, w):
    N, H = x.shape
    xf = x.reshape(N * H)
    return _make_gather(N, H, w.shape[0], w.shape[1])(xf, w)
